# baseline (speedup 1.0000x reference)
import functools

import jax
import jax.numpy as jnp
from jax import lax
from jax.experimental import pallas as pl
from jax.experimental.pallas import tpu as pltpu, tpu_sc as plsc

_EPS_BN = 1e-5
_NB1 = 2048
_NB2 = 2048
_CH = 128


def _k1a_body(x1_ref, x2_ref, prm_ref, idx_ref, wts_ref, *, nb, m, mm):
    del prm_ref
    x1 = x1_ref[0]
    x2 = x2_ref[0]
    aa = jnp.sum(x1 * x1, axis=1, keepdims=True)
    bb = jnp.sum(x2 * x2, axis=1)
    ab2 = jax.lax.dot_general(x1, -2.0 * x2, (((1,), (1,)), ((), ())),
                              preferred_element_type=jnp.float32)
    sq = jnp.maximum((ab2 + aa) + bb[None, :], 0.0)

    iota = lax.broadcasted_iota(jnp.int32, (nb, m), 1)
    inf = jnp.float32(jnp.inf)
    dc = sq
    mins, idxs = [], []
    for _ in range(3):
        mk = jnp.min(dc, axis=1, keepdims=True)
        ik = jnp.min(jnp.where(dc == mk, iota, m), axis=1, keepdims=True)
        mins.append(mk)
        idxs.append(ik)
        dc = jnp.where(iota == ik, inf, dc)

    r = [1.0 / (jnp.sqrt(mk) + 1e-8) for mk in mins]
    norm = r[0] + r[1] + r[2]
    base = pl.program_id(0) * mm
    zi = jnp.zeros((5, nb), jnp.int32)
    idx_ref[0] = jnp.concatenate(
        [(idxs[0] + base).T, (idxs[1] + base).T, (idxs[2] + base).T, zi], 0)
    wts_ref[0] = jnp.concatenate(
        [jnp.broadcast_to(r[0] / norm, (nb, 16)),
         jnp.broadcast_to(r[1] / norm, (nb, 16)),
         jnp.broadcast_to(r[2] / norm, (nb, 16))], 1)


def _sc_gather(table, idx, wts, *, nbn, c2):
    info = plsc.get_sparse_core_info()
    nw = info.num_cores * info.num_subcores
    ppw = nbn // nw
    nch = ppw // _CH
    mesh = plsc.VectorSubcoreMesh(core_axis_name="c", subcore_axis_name="s")

    @functools.partial(
        pl.kernel, mesh=mesh,
        out_type=jax.ShapeDtypeStruct((nbn, c2), jnp.float32),
        scratch_types=[
            pltpu.VMEM((_CH,), jnp.int32),
            pltpu.VMEM((_CH,), jnp.int32),
            pltpu.VMEM((_CH,), jnp.int32),
            pltpu.VMEM((_CH, 48), jnp.float32),
            pltpu.VMEM((_CH, c2), jnp.float32),
            pltpu.VMEM((_CH, c2), jnp.float32),
            pltpu.VMEM((_CH, c2), jnp.float32),
            pltpu.VMEM((_CH, c2), jnp.float32),
            pltpu.SemaphoreType.DMA,
        ],
    )
    def k(table_hbm, idx_hbm, wts_hbm, out_hbm,
          i1_v, i2_v, i3_v, w_v, r1_v, r2_v, r3_v, o_v, sem):
        wid = lax.axis_index("s") * info.num_cores + lax.axis_index("c")
        wbase = wid * ppw

        def chunk_body(c, _):
            base = wbase + c * _CH
            pltpu.sync_copy(idx_hbm.at[0, pl.ds(base, _CH)], i1_v)
            pltpu.sync_copy(idx_hbm.at[1, pl.ds(base, _CH)], i2_v)
            pltpu.sync_copy(idx_hbm.at[2, pl.ds(base, _CH)], i3_v)
            pltpu.sync_copy(wts_hbm.at[pl.ds(base, _CH)], w_v)
            pltpu.async_copy(table_hbm.at[i1_v], r1_v, sem).wait()
            pltpu.async_copy(table_hbm.at[i2_v], r2_v, sem).wait()
            pltpu.async_copy(table_hbm.at[i3_v], r3_v, sem).wait()

            def pt_body(p, _):
                w1 = w_v[p, pl.ds(0, 16)]
                w2 = w_v[p, pl.ds(16, 16)]
                w3 = w_v[p, pl.ds(32, 16)]
                for j in range(c2 // 16):
                    sl = pl.ds(j * 16, 16)
                    o_v[p, sl] = (w1 * r1_v[p, sl] + w2 * r2_v[p, sl]
                                  + w3 * r3_v[p, sl])
                return 0

            lax.fori_loop(0, _CH, pt_body, 0)
            pltpu.sync_copy(o_v, out_hbm.at[pl.ds(base, _CH)])
            return 0

        lax.fori_loop(0, nch, chunk_body, 0)

    return k(table, idx, wts)


def _k1b_body(p1_ref, it_ref, w1at_ref, w1bt_ref, prm_ref, y1_ref, st_ref):
    p1 = p1_ref[0].T
    y1 = (jnp.dot(p1, w1at_ref[...], preferred_element_type=jnp.float32)
          + jnp.dot(it_ref[0], w1bt_ref[...],
                    preferred_element_type=jnp.float32)
          + prm_ref[0:1, :])
    y1_ref[0] = y1
    s = jnp.sum(y1, axis=0, keepdims=True)
    ss = jnp.sum(y1 * y1, axis=0, keepdims=True)
    upd = jnp.concatenate(
        [s, ss, jnp.zeros((6, s.shape[1]), jnp.float32)], axis=0)

    @pl.when(jnp.logical_and(pl.program_id(0) == 0, pl.program_id(1) == 0))
    def _init():
        st_ref[...] = jnp.zeros_like(st_ref)

    st_ref[...] += upd


def _k2_body(y1_ref, st_ref, w2t_ref, prm_ref, y2_ref, st2_ref, *, cnt):
    mean = st_ref[0:1, :] / cnt
    var = st_ref[1:2, :] / cnt - mean * mean
    inv = 1.0 / jnp.sqrt(var + _EPS_BN)
    g1 = prm_ref[0:1, :]
    be1 = prm_ref[1:2, :]
    b2 = prm_ref[2:3, :]
    z = jnp.maximum((y1_ref[0] - mean) * inv * g1 + be1, 0.0)
    y2 = jnp.dot(z, w2t_ref[...], preferred_element_type=jnp.float32) + b2
    y2_ref[0] = y2
    s = jnp.sum(y2, axis=0, keepdims=True)
    ss = jnp.sum(y2 * y2, axis=0, keepdims=True)
    upd = jnp.concatenate(
        [s, ss, jnp.zeros((6, s.shape[1]), jnp.float32)], axis=0)

    @pl.when(jnp.logical_and(pl.program_id(0) == 0, pl.program_id(1) == 0))
    def _init():
        st2_ref[...] = jnp.zeros_like(st2_ref)

    st2_ref[...] += upd


def _k3_body(y2_ref, st_ref, prm_ref, out_ref, *, cnt):
    mean = st_ref[0:1, :] / cnt
    var = st_ref[1:2, :] / cnt - mean * mean
    inv = 1.0 / jnp.sqrt(var + _EPS_BN)
    g2 = prm_ref[0:1, :]
    be2 = prm_ref[1:2, :]
    z = jnp.maximum((y2_ref[0] - mean) * inv * g2 + be2, 0.0)
    out_ref[0] = z.T


def kernel(xyz1, xyz2, points1, points2, W1, b1, g1, be1, W2, b2, g2, be2):
    B, N, _ = xyz1.shape
    M = xyz2.shape[1]
    C1 = points1.shape[1]
    C2 = points2.shape[1]
    H1 = W1.shape[0]
    H2 = W2.shape[0]
    nb1 = _NB1 if N % _NB1 == 0 else N
    nb2 = _NB2 if N % _NB2 == 0 else N
    cnt = float(B * N)

    pts = jnp.transpose(points2, (0, 2, 1))
    table = jnp.reshape(pts, (B * M, C2))
    w1at = jnp.transpose(W1[:, :C1])
    w1bt = jnp.transpose(W1[:, C1:])
    w2t = jnp.transpose(W2)
    prm1 = jnp.concatenate([b1[None, :], jnp.zeros((7, H1), jnp.float32)], 0)
    prm2 = jnp.concatenate(
        [g1[None, :], be1[None, :], b2[None, :],
         jnp.zeros((5, H1), jnp.float32)], 0)
    prm3 = jnp.concatenate(
        [g2[None, :], be2[None, :], jnp.zeros((6, H2), jnp.float32)], 0)

    grid1 = (B, N // nb1)
    grid2 = (B, N // nb2)

    idx, wts = pl.pallas_call(
        functools.partial(_k1a_body, nb=nb1, m=M, mm=M),
        grid=grid1,
        in_specs=[
            pl.BlockSpec((1, nb1, 3), lambda b, i: (b, i, 0)),
            pl.BlockSpec((1, M, 3), lambda b, i: (b, 0, 0)),
            pl.BlockSpec((8, H1), lambda b, i: (0, 0)),
        ],
        out_specs=[
            pl.BlockSpec((1, 8, nb1), lambda b, i: (b, 0, i)),
            pl.BlockSpec((1, nb1, 48), lambda b, i: (b, i, 0)),
        ],
        out_shape=[
            jax.ShapeDtypeStruct((B, 8, N), jnp.int32),
            jax.ShapeDtypeStruct((B, N, 48), jnp.float32),
        ],
    )(xyz1, xyz2, prm1)

    idxf = jnp.reshape(jnp.transpose(idx, (1, 0, 2)), (8, B * N))
    wtsf = jnp.reshape(wts, (B * N, 48))

    interp = _sc_gather(table, idxf, wtsf, nbn=B * N, c2=C2)
    interp = jnp.reshape(interp, (B, N, C2))

    y1, st1 = pl.pallas_call(
        _k1b_body,
        grid=grid2,
        in_specs=[
            pl.BlockSpec((1, C1, nb2), lambda b, i: (b, 0, i)),
            pl.BlockSpec((1, nb2, C2), lambda b, i: (b, i, 0)),
            pl.BlockSpec((C1, H1), lambda b, i: (0, 0)),
            pl.BlockSpec((C2, H1), lambda b, i: (0, 0)),
            pl.BlockSpec((8, H1), lambda b, i: (0, 0)),
        ],
        out_specs=[
            pl.BlockSpec((1, nb2, H1), lambda b, i: (b, i, 0)),
            pl.BlockSpec((8, H1), lambda b, i: (0, 0)),
        ],
        out_shape=[
            jax.ShapeDtypeStruct((B, N, H1), jnp.float32),
            jax.ShapeDtypeStruct((8, H1), jnp.float32),
        ],
    )(points1, interp, w1at, w1bt, prm1)

    y2, st2 = pl.pallas_call(
        functools.partial(_k2_body, cnt=cnt),
        grid=grid2,
        in_specs=[
            pl.BlockSpec((1, nb2, H1), lambda b, i: (b, i, 0)),
            pl.BlockSpec((8, H1), lambda b, i: (0, 0)),
            pl.BlockSpec((H1, H2), lambda b, i: (0, 0)),
            pl.BlockSpec((8, H1), lambda b, i: (0, 0)),
        ],
        out_specs=[
            pl.BlockSpec((1, nb2, H2), lambda b, i: (b, i, 0)),
            pl.BlockSpec((8, H2), lambda b, i: (0, 0)),
        ],
        out_shape=[
            jax.ShapeDtypeStruct((B, N, H2), jnp.float32),
            jax.ShapeDtypeStruct((8, H2), jnp.float32),
        ],
    )(y1, st1, w2t, prm2)

    out = pl.pallas_call(
        functools.partial(_k3_body, cnt=cnt),
        grid=grid2,
        in_specs=[
            pl.BlockSpec((1, nb2, H2), lambda b, i: (b, i, 0)),
            pl.BlockSpec((8, H2), lambda b, i: (0, 0)),
            pl.BlockSpec((8, H2), lambda b, i: (0, 0)),
        ],
        out_specs=pl.BlockSpec((1, H2, nb2), lambda b, i: (b, 0, i)),
        out_shape=jax.ShapeDtypeStruct((B, H2, N), jnp.float32),
    )(y2, st2, prm3)

    return out


# speedup vs baseline: 1.3527x; 1.3527x over previous
import functools

import jax
import jax.numpy as jnp
from jax.experimental import pallas as pl

_EPS_BN = 1e-5
_NB1 = 2048
_NB2 = 2048


def _k1_body(x1_ref, x2_ref, p1_ref, pts_ref, w1at_ref, w1bt_ref, prm_ref,
             y1_ref, st_ref, *, nb, m):
    x1 = x1_ref[0]
    x2 = x2_ref[0]
    aa = jnp.sum(x1 * x1, axis=1, keepdims=True)
    bb = jnp.sum(x2 * x2, axis=1)
    ab2 = jax.lax.dot_general(x1, -2.0 * x2, (((1,), (1,)), ((), ())),
                              preferred_element_type=jnp.float32)
    sq = jnp.maximum((ab2 + aa) + bb[None, :], 0.0)

    iota = jax.lax.broadcasted_iota(jnp.int32, (nb, m), 1)
    inf = jnp.float32(jnp.inf)
    dc = sq
    mins, idxs = [], []
    for _ in range(3):
        mk = jnp.min(dc, axis=1, keepdims=True)
        ik = jnp.min(jnp.where(dc == mk, iota, m), axis=1, keepdims=True)
        mins.append(mk)
        idxs.append(ik)
        dc = jnp.where(iota == ik, inf, dc)

    r = [1.0 / (jnp.sqrt(mk) + 1e-8) for mk in mins]
    norm = r[0] + r[1] + r[2]
    oh = jnp.zeros((nb, m), jnp.float32)
    for k in range(3):
        oh = oh + jnp.where(iota == idxs[k], r[k] / norm, 0.0)

    interp = jnp.dot(oh, pts_ref[0], preferred_element_type=jnp.float32)
    p1 = p1_ref[0].T
    y1 = (jnp.dot(p1, w1at_ref[...], preferred_element_type=jnp.float32)
          + jnp.dot(interp, w1bt_ref[...], preferred_element_type=jnp.float32)
          + prm_ref[0:1, :])
    y1_ref[0] = y1.astype(jnp.bfloat16)

    s = jnp.sum(y1, axis=0, keepdims=True)
    ss = jnp.sum(y1 * y1, axis=0, keepdims=True)
    upd = jnp.concatenate(
        [s, ss, jnp.zeros((6, s.shape[1]), jnp.float32)], axis=0)

    @pl.when(jnp.logical_and(pl.program_id(0) == 0, pl.program_id(1) == 0))
    def _init():
        st_ref[...] = jnp.zeros_like(st_ref)

    st_ref[...] += upd


def _k2_body(y1_ref, st_ref, w2t_ref, prm_ref, y2_ref, st2_ref, *, cnt):
    mean = st_ref[0:1, :] / cnt
    var = st_ref[1:2, :] / cnt - mean * mean
    inv = 1.0 / jnp.sqrt(var + _EPS_BN)
    g1 = prm_ref[0:1, :]
    be1 = prm_ref[1:2, :]
    b2 = prm_ref[2:3, :]
    y1 = y1_ref[0].astype(jnp.float32)
    z = jnp.maximum((y1 - mean) * inv * g1 + be1, 0.0)
    y2 = jnp.dot(z, w2t_ref[...], preferred_element_type=jnp.float32) + b2
    y2_ref[0] = y2.astype(jnp.bfloat16)

    s = jnp.sum(y2, axis=0, keepdims=True)
    ss = jnp.sum(y2 * y2, axis=0, keepdims=True)
    upd = jnp.concatenate(
        [s, ss, jnp.zeros((6, s.shape[1]), jnp.float32)], axis=0)

    @pl.when(jnp.logical_and(pl.program_id(0) == 0, pl.program_id(1) == 0))
    def _init():
        st2_ref[...] = jnp.zeros_like(st2_ref)

    st2_ref[...] += upd


def _k3_body(y2_ref, st_ref, prm_ref, out_ref, *, cnt):
    mean = st_ref[0:1, :] / cnt
    var = st_ref[1:2, :] / cnt - mean * mean
    inv = 1.0 / jnp.sqrt(var + _EPS_BN)
    g2 = prm_ref[0:1, :]
    be2 = prm_ref[1:2, :]
    y2 = y2_ref[0].astype(jnp.float32)
    z = jnp.maximum((y2 - mean) * inv * g2 + be2, 0.0)
    out_ref[0] = z.T


def kernel(xyz1, xyz2, points1, points2, W1, b1, g1, be1, W2, b2, g2, be2):
    B, N, _ = xyz1.shape
    M = xyz2.shape[1]
    C1 = points1.shape[1]
    H1 = W1.shape[0]
    H2 = W2.shape[0]
    nb1 = _NB1 if N % _NB1 == 0 else N
    nb2 = _NB2 if N % _NB2 == 0 else N
    cnt = float(B * N)

    pts = jnp.transpose(points2, (0, 2, 1))
    w1at = jnp.transpose(W1[:, :C1])
    w1bt = jnp.transpose(W1[:, C1:])
    w2t = jnp.transpose(W2)
    prm1 = jnp.concatenate([b1[None, :], jnp.zeros((7, H1), jnp.float32)], 0)
    prm2 = jnp.concatenate(
        [g1[None, :], be1[None, :], b2[None, :],
         jnp.zeros((5, H1), jnp.float32)], 0)
    prm3 = jnp.concatenate(
        [g2[None, :], be2[None, :], jnp.zeros((6, H2), jnp.float32)], 0)

    grid1 = (B, N // nb1)
    grid2 = (B, N // nb2)

    y1, st1 = pl.pallas_call(
        functools.partial(_k1_body, nb=nb1, m=M),
        grid=grid1,
        in_specs=[
            pl.BlockSpec((1, nb1, 3), lambda b, i: (b, i, 0)),
            pl.BlockSpec((1, M, 3), lambda b, i: (b, 0, 0)),
            pl.BlockSpec((1, C1, nb1), lambda b, i: (b, 0, i)),
            pl.BlockSpec((1, M, points2.shape[1]), lambda b, i: (b, 0, 0)),
            pl.BlockSpec((C1, H1), lambda b, i: (0, 0)),
            pl.BlockSpec((points2.shape[1], H1), lambda b, i: (0, 0)),
            pl.BlockSpec((8, H1), lambda b, i: (0, 0)),
        ],
        out_specs=[
            pl.BlockSpec((1, nb1, H1), lambda b, i: (b, i, 0)),
            pl.BlockSpec((8, H1), lambda b, i: (0, 0)),
        ],
        out_shape=[
            jax.ShapeDtypeStruct((B, N, H1), jnp.bfloat16),
            jax.ShapeDtypeStruct((8, H1), jnp.float32),
        ],
    )(xyz1, xyz2, points1, pts, w1at, w1bt, prm1)

    y2, st2 = pl.pallas_call(
        functools.partial(_k2_body, cnt=cnt),
        grid=grid2,
        in_specs=[
            pl.BlockSpec((1, nb2, H1), lambda b, i: (b, i, 0)),
            pl.BlockSpec((8, H1), lambda b, i: (0, 0)),
            pl.BlockSpec((H1, H2), lambda b, i: (0, 0)),
            pl.BlockSpec((8, H1), lambda b, i: (0, 0)),
        ],
        out_specs=[
            pl.BlockSpec((1, nb2, H2), lambda b, i: (b, i, 0)),
            pl.BlockSpec((8, H2), lambda b, i: (0, 0)),
        ],
        out_shape=[
            jax.ShapeDtypeStruct((B, N, H2), jnp.bfloat16),
            jax.ShapeDtypeStruct((8, H2), jnp.float32),
        ],
    )(y1, st1, w2t, prm2)

    out = pl.pallas_call(
        functools.partial(_k3_body, cnt=cnt),
        grid=grid2,
        in_specs=[
            pl.BlockSpec((1, nb2, H2), lambda b, i: (b, i, 0)),
            pl.BlockSpec((8, H2), lambda b, i: (0, 0)),
            pl.BlockSpec((8, H2), lambda b, i: (0, 0)),
        ],
        out_specs=pl.BlockSpec((1, H2, nb2), lambda b, i: (b, 0, i)),
        out_shape=jax.ShapeDtypeStruct((B, H2, N), jnp.float32),
    )(y2, st2, prm3)

    return out


# speedup vs baseline: 1.4006x; 1.0354x over previous
import functools

import jax
import jax.numpy as jnp
from jax.experimental import pallas as pl

_EPS_BN = 1e-5
_NB1 = 2048
_NB2 = 4096


def _k1_body(x1_ref, x2_ref, p1_ref, pts_ref, w1at_ref, w1bt_ref, prm_ref,
             y1_ref, st_ref, *, nb, m):
    x1 = x1_ref[0]
    x2 = x2_ref[0]
    aa = jnp.sum(x1 * x1, axis=1, keepdims=True)
    bb = jnp.sum(x2 * x2, axis=1)
    ab2 = jax.lax.dot_general(x1, -2.0 * x2, (((1,), (1,)), ((), ())),
                              preferred_element_type=jnp.float32)
    sq = jnp.maximum((ab2 + aa) + bb[None, :], 0.0)

    iota = jax.lax.broadcasted_iota(jnp.int32, (nb, m), 1)
    inf = jnp.float32(jnp.inf)
    dc = sq
    mins, idxs = [], []
    for _ in range(3):
        mk = jnp.min(dc, axis=1, keepdims=True)
        ik = jnp.min(jnp.where(dc == mk, iota, m), axis=1, keepdims=True)
        mins.append(mk)
        idxs.append(ik)
        dc = jnp.where(iota == ik, inf, dc)

    r = [1.0 / (jnp.sqrt(mk) + 1e-8) for mk in mins]
    norm = r[0] + r[1] + r[2]
    oh = jnp.zeros((nb, m), jnp.float32)
    for k in range(3):
        oh = oh + jnp.where(iota == idxs[k], r[k] / norm, 0.0)

    interp = jnp.dot(oh, pts_ref[0], preferred_element_type=jnp.float32)
    p1 = p1_ref[0].T
    y1 = (jnp.dot(p1, w1at_ref[...], preferred_element_type=jnp.float32)
          + jnp.dot(interp, w1bt_ref[...], preferred_element_type=jnp.float32)
          + prm_ref[0:1, :])
    y1_ref[0] = y1.astype(jnp.bfloat16)

    s = jnp.sum(y1, axis=0, keepdims=True)
    ss = jnp.sum(y1 * y1, axis=0, keepdims=True)
    upd = jnp.concatenate(
        [s, ss, jnp.zeros((6, s.shape[1]), jnp.float32)], axis=0)

    @pl.when(jnp.logical_and(pl.program_id(0) == 0, pl.program_id(1) == 0))
    def _init():
        st_ref[...] = jnp.zeros_like(st_ref)

    st_ref[...] += upd


def _k2_body(y1_ref, st_ref, w2t_ref, prm_ref, y2_ref, st2_ref, *, cnt):
    mean = st_ref[0:1, :] / cnt
    var = st_ref[1:2, :] / cnt - mean * mean
    inv = 1.0 / jnp.sqrt(var + _EPS_BN)
    g1 = prm_ref[0:1, :]
    be1 = prm_ref[1:2, :]
    b2 = prm_ref[2:3, :]
    y1 = y1_ref[0].astype(jnp.float32)
    z = jnp.maximum((y1 - mean) * inv * g1 + be1, 0.0)
    y2 = jnp.dot(z, w2t_ref[...], preferred_element_type=jnp.float32) + b2
    y2_ref[0] = y2.astype(jnp.bfloat16)

    s = jnp.sum(y2, axis=0, keepdims=True)
    ss = jnp.sum(y2 * y2, axis=0, keepdims=True)
    upd = jnp.concatenate(
        [s, ss, jnp.zeros((6, s.shape[1]), jnp.float32)], axis=0)

    @pl.when(jnp.logical_and(pl.program_id(0) == 0, pl.program_id(1) == 0))
    def _init():
        st2_ref[...] = jnp.zeros_like(st2_ref)

    st2_ref[...] += upd


def _k3_body(y2_ref, st_ref, prm_ref, out_ref, *, cnt):
    mean = st_ref[0:1, :] / cnt
    var = st_ref[1:2, :] / cnt - mean * mean
    inv = 1.0 / jnp.sqrt(var + _EPS_BN)
    g2 = prm_ref[0:1, :]
    be2 = prm_ref[1:2, :]
    y2 = y2_ref[0].astype(jnp.float32)
    z = jnp.maximum((y2 - mean) * inv * g2 + be2, 0.0)
    out_ref[0] = z.T


def kernel(xyz1, xyz2, points1, points2, W1, b1, g1, be1, W2, b2, g2, be2):
    B, N, _ = xyz1.shape
    M = xyz2.shape[1]
    C1 = points1.shape[1]
    H1 = W1.shape[0]
    H2 = W2.shape[0]
    nb1 = _NB1 if N % _NB1 == 0 else N
    nb2 = _NB2 if N % _NB2 == 0 else N
    cnt = float(B * N)

    pts = jnp.transpose(points2, (0, 2, 1))
    w1at = jnp.transpose(W1[:, :C1])
    w1bt = jnp.transpose(W1[:, C1:])
    w2t = jnp.transpose(W2)
    prm1 = jnp.concatenate([b1[None, :], jnp.zeros((7, H1), jnp.float32)], 0)
    prm2 = jnp.concatenate(
        [g1[None, :], be1[None, :], b2[None, :],
         jnp.zeros((5, H1), jnp.float32)], 0)
    prm3 = jnp.concatenate(
        [g2[None, :], be2[None, :], jnp.zeros((6, H2), jnp.float32)], 0)

    grid1 = (B, N // nb1)
    grid2 = (B, N // nb2)

    y1, st1 = pl.pallas_call(
        functools.partial(_k1_body, nb=nb1, m=M),
        grid=grid1,
        in_specs=[
            pl.BlockSpec((1, nb1, 3), lambda b, i: (b, i, 0)),
            pl.BlockSpec((1, M, 3), lambda b, i: (b, 0, 0)),
            pl.BlockSpec((1, C1, nb1), lambda b, i: (b, 0, i)),
            pl.BlockSpec((1, M, points2.shape[1]), lambda b, i: (b, 0, 0)),
            pl.BlockSpec((C1, H1), lambda b, i: (0, 0)),
            pl.BlockSpec((points2.shape[1], H1), lambda b, i: (0, 0)),
            pl.BlockSpec((8, H1), lambda b, i: (0, 0)),
        ],
        out_specs=[
            pl.BlockSpec((1, nb1, H1), lambda b, i: (b, i, 0)),
            pl.BlockSpec((8, H1), lambda b, i: (0, 0)),
        ],
        out_shape=[
            jax.ShapeDtypeStruct((B, N, H1), jnp.bfloat16),
            jax.ShapeDtypeStruct((8, H1), jnp.float32),
        ],
    )(xyz1, xyz2, points1, pts, w1at, w1bt, prm1)

    y2, st2 = pl.pallas_call(
        functools.partial(_k2_body, cnt=cnt),
        grid=grid2,
        in_specs=[
            pl.BlockSpec((1, nb2, H1), lambda b, i: (b, i, 0)),
            pl.BlockSpec((8, H1), lambda b, i: (0, 0)),
            pl.BlockSpec((H1, H2), lambda b, i: (0, 0)),
            pl.BlockSpec((8, H1), lambda b, i: (0, 0)),
        ],
        out_specs=[
            pl.BlockSpec((1, nb2, H2), lambda b, i: (b, i, 0)),
            pl.BlockSpec((8, H2), lambda b, i: (0, 0)),
        ],
        out_shape=[
            jax.ShapeDtypeStruct((B, N, H2), jnp.bfloat16),
            jax.ShapeDtypeStruct((8, H2), jnp.float32),
        ],
    )(y1, st1, w2t, prm2)

    out = pl.pallas_call(
        functools.partial(_k3_body, cnt=cnt),
        grid=grid2,
        in_specs=[
            pl.BlockSpec((1, nb2, H2), lambda b, i: (b, i, 0)),
            pl.BlockSpec((8, H2), lambda b, i: (0, 0)),
            pl.BlockSpec((8, H2), lambda b, i: (0, 0)),
        ],
        out_specs=pl.BlockSpec((1, H2, nb2), lambda b, i: (b, 0, i)),
        out_shape=jax.ShapeDtypeStruct((B, H2, N), jnp.float32),
    )(y2, st2, prm3)

    return out
